# Initial kernel scaffold; baseline (speedup 1.0000x reference)
import jax
import jax.numpy as jnp
from jax.experimental import pallas as pl


def kernel(x, Wc, bc, Wp, bp, gates):
    raise NotImplementedError("write your pallas kernel here")


# speedup vs baseline: 1.1952x; 1.1952x over previous
import functools

import jax
import jax.numpy as jnp
from jax.experimental import pallas as pl
from jax.experimental.pallas import tpu as pltpu

E = 8
TOP = 2
C = 192
G = 4
B = 8
H = 16
W = 16
CCAP = 192
HW = H * W
BHW = B * HW


def _shift_hw(x4, sh, sw):
    if sh > 0:
        x4 = jnp.concatenate([x4[:, sh:], jnp.zeros_like(x4[:, :sh])], axis=1)
    elif sh < 0:
        x4 = jnp.concatenate([jnp.zeros_like(x4[:, sh:]), x4[:, :sh]], axis=1)
    if sw > 0:
        x4 = jnp.concatenate([x4[:, :, sw:], jnp.zeros_like(x4[:, :, :sw])], axis=2)
    elif sw < 0:
        x4 = jnp.concatenate([jnp.zeros_like(x4[:, :, sw:]), x4[:, :, :sw]], axis=2)
    return x4


def _moe_body(x_ref, gates_ref, wc_ref, bc_ref, wp_ref, bp_ref,
              ys_ref, loss_ref, cw_ref):
    e = pl.program_id(0)

    @pl.when(e == 0)
    def _gating():
        x_gap = jnp.mean(x_ref[...], axis=1)
        eio = jax.lax.broadcasted_iota(jnp.int32, (B, E), 1)
        loss_acc = jnp.float32(0.0)
        for g in range(G):
            logits = jnp.dot(x_gap, gates_ref[g], preferred_element_type=jnp.float32)
            m = jnp.max(logits, axis=1, keepdims=True)
            ex = jnp.exp(logits - m)
            probs = ex / jnp.sum(ex, axis=1, keepdims=True)
            usage = jnp.sum(probs, axis=0)
            mu = jnp.mean(usage)
            var = jnp.mean((usage - mu) ** 2)
            loss_acc = loss_acc + var / (mu * mu + 1e-10)
            v1 = jnp.max(probs, axis=1, keepdims=True)
            i1 = jnp.min(jnp.where(probs == v1, eio, E + 1), axis=1, keepdims=True)
            p2 = jnp.where(eio == i1, -1.0, probs)
            v2 = jnp.max(p2, axis=1, keepdims=True)
            i2 = jnp.min(jnp.where(p2 == v2, eio, E + 1), axis=1, keepdims=True)
            t = jnp.exp(v2 - v1)
            w1 = 1.0 / (1.0 + t)
            w2 = t / (1.0 + t)
            for e_ in range(E):
                cw_ref[g, e_] = jnp.where(i1 == e_, w1,
                                          jnp.where(i2 == e_, w2, 0.0))
        loss_ref[...] = jnp.broadcast_to(loss_acc / G, (1, 1))
        ys_ref[...] = jnp.zeros((G, BHW, C), jnp.float32)

    x4 = x_ref[...].reshape(B, H, W, C)
    acc = jnp.zeros((BHW, CCAP), jnp.float32)
    for dy in range(3):
        for dx in range(3):
            xs = _shift_hw(x4, dy - 1, dx - 1).reshape(BHW, C)
            acc = acc + jnp.dot(xs, wc_ref[0, dy, dx],
                                preferred_element_type=jnp.float32)
    u = acc + bc_ref[0]
    sn = jnp.sum(u * u, axis=1, keepdims=True)
    u = (sn / (1.0 + sn)) * u / (jnp.sqrt(sn) + 1e-8)
    out2d = jnp.dot(u, wp_ref[0], preferred_element_type=jnp.float32) + bp_ref[0]

    rb = jax.lax.broadcasted_iota(jnp.int32, (BHW, B), 0) // HW
    cb = jax.lax.broadcasted_iota(jnp.int32, (BHW, B), 1)
    oh = (rb == cb).astype(jnp.float32)
    for g in range(G):
        wsel = cw_ref[g, e]
        wrow = jnp.dot(oh, wsel, preferred_element_type=jnp.float32)
        ys_ref[g] = ys_ref[g] + wrow * out2d


@jax.jit
def _moe(x, Wc, bc, Wp, bp, gates):
    x3 = jnp.transpose(x, (0, 2, 3, 1)).reshape(B, HW, C)
    Wc_r = jnp.transpose(Wc, (0, 3, 4, 2, 1))
    bc_r = bc.reshape(E, 1, CCAP)
    Wp_r = jnp.transpose(Wp[..., 0, 0], (0, 2, 1))
    bp_r = bp.reshape(E, 1, C)

    ys, loss = pl.pallas_call(
        _moe_body,
        grid=(E,),
        in_specs=[
            pl.BlockSpec((B, HW, C), lambda e: (0, 0, 0)),
            pl.BlockSpec((G, C, E), lambda e: (0, 0, 0)),
            pl.BlockSpec((1, 3, 3, C, CCAP), lambda e: (e, 0, 0, 0, 0)),
            pl.BlockSpec((1, 1, CCAP), lambda e: (e, 0, 0)),
            pl.BlockSpec((1, CCAP, C), lambda e: (e, 0, 0)),
            pl.BlockSpec((1, 1, C), lambda e: (e, 0, 0)),
        ],
        out_specs=[
            pl.BlockSpec((G, BHW, C), lambda e: (0, 0, 0)),
            pl.BlockSpec((1, 1), lambda e: (0, 0)),
        ],
        out_shape=[
            jax.ShapeDtypeStruct((G, BHW, C), jnp.float32),
            jax.ShapeDtypeStruct((1, 1), jnp.float32),
        ],
        scratch_shapes=[pltpu.VMEM((G, E, B, 1), jnp.float32)],
        compiler_params=pltpu.CompilerParams(
            dimension_semantics=("arbitrary",),
        ),
    )(x3, gates, Wc_r, bc_r, Wp_r, bp_r)

    ys4 = jnp.transpose(ys.reshape(G, B, H, W, C), (0, 1, 4, 2, 3))
    return ys4[0], ys4[1], ys4[2], ys4[3], loss[0, 0]


def kernel(x, Wc, bc, Wp, bp, gates):
    return _moe(x, Wc, bc, Wp, bp, gates)


# speedup vs baseline: 2.2091x; 1.8483x over previous
import functools

import jax
import jax.numpy as jnp
from jax.experimental import pallas as pl
from jax.experimental.pallas import tpu as pltpu

E = 8
TOP = 2
C = 192
G = 4
B = 8
H = 16
W = 16
CCAP = 192
HW = H * W
BHW = B * HW


def _shift_hw(x4, sh, sw):
    if sh > 0:
        x4 = jnp.concatenate([x4[:, sh:], jnp.zeros_like(x4[:, :sh])], axis=1)
    elif sh < 0:
        x4 = jnp.concatenate([jnp.zeros_like(x4[:, sh:]), x4[:, :sh]], axis=1)
    if sw > 0:
        x4 = jnp.concatenate([x4[:, :, sw:], jnp.zeros_like(x4[:, :, :sw])], axis=2)
    elif sw < 0:
        x4 = jnp.concatenate([jnp.zeros_like(x4[:, :, sw:]), x4[:, :, :sw]], axis=2)
    return x4


def _moe_body(x_ref, xb_ref, gates_ref, wc_ref, bc_ref, wp_ref, bp_ref,
              ys_ref, loss_ref, cw_ref):
    e = pl.program_id(0)

    @pl.when(e == 0)
    def _gating():
        x_gap = jnp.mean(x_ref[...], axis=1)
        eio = jax.lax.broadcasted_iota(jnp.int32, (B, E), 1)
        loss_acc = jnp.float32(0.0)
        for g in range(G):
            logits = jnp.dot(x_gap, gates_ref[g], preferred_element_type=jnp.float32)
            m = jnp.max(logits, axis=1, keepdims=True)
            ex = jnp.exp(logits - m)
            probs = ex / jnp.sum(ex, axis=1, keepdims=True)
            usage = jnp.sum(probs, axis=0)
            mu = jnp.mean(usage)
            var = jnp.mean((usage - mu) ** 2)
            loss_acc = loss_acc + var / (mu * mu + 1e-10)
            v1 = jnp.max(probs, axis=1, keepdims=True)
            i1 = jnp.min(jnp.where(probs == v1, eio, E + 1), axis=1, keepdims=True)
            p2 = jnp.where(eio == i1, -1.0, probs)
            v2 = jnp.max(p2, axis=1, keepdims=True)
            i2 = jnp.min(jnp.where(p2 == v2, eio, E + 1), axis=1, keepdims=True)
            t = jnp.exp(v2 - v1)
            w1 = 1.0 / (1.0 + t)
            w2 = t / (1.0 + t)
            for e_ in range(E):
                cw_ref[g, e_] = jnp.where(i1 == e_, w1,
                                          jnp.where(i2 == e_, w2, 0.0))
        loss_ref[...] = jnp.broadcast_to(loss_acc / G, (1, 1))
        ys_ref[...] = jnp.zeros((G, BHW, C), jnp.float32)

    x4 = xb_ref[...].reshape(B, H, W, C)
    acc = jnp.zeros((BHW, CCAP), jnp.float32)
    for dy in range(3):
        for dx in range(3):
            xs = _shift_hw(x4, dy - 1, dx - 1).reshape(BHW, C)
            acc = acc + jnp.dot(xs, wc_ref[0, dy, dx],
                                preferred_element_type=jnp.float32)
    u = acc + bc_ref[0]
    sn = jnp.sum(u * u, axis=1, keepdims=True)
    u = (sn / (1.0 + sn)) * u / (jnp.sqrt(sn) + 1e-8)
    out2d = jnp.dot(u.astype(jnp.bfloat16), wp_ref[0],
                    preferred_element_type=jnp.float32) + bp_ref[0]

    rb = jax.lax.broadcasted_iota(jnp.int32, (BHW, B), 0) // HW
    cb = jax.lax.broadcasted_iota(jnp.int32, (BHW, B), 1)
    oh = (rb == cb).astype(jnp.float32)
    for g in range(G):
        wsel = cw_ref[g, e]
        wrow = jnp.dot(oh, wsel, preferred_element_type=jnp.float32)
        ys_ref[g] = ys_ref[g] + wrow * out2d


@jax.jit
def _moe(x, Wc, bc, Wp, bp, gates):
    x3 = jnp.transpose(x, (0, 2, 3, 1)).reshape(B, HW, C)
    x3b = x3.astype(jnp.bfloat16)
    Wc_r = jnp.transpose(Wc, (0, 3, 4, 2, 1)).astype(jnp.bfloat16)
    bc_r = bc.reshape(E, 1, CCAP)
    Wp_r = jnp.transpose(Wp[..., 0, 0], (0, 2, 1)).astype(jnp.bfloat16)
    bp_r = bp.reshape(E, 1, C)

    ys, loss = pl.pallas_call(
        _moe_body,
        grid=(E,),
        in_specs=[
            pl.BlockSpec((B, HW, C), lambda e: (0, 0, 0)),
            pl.BlockSpec((B, HW, C), lambda e: (0, 0, 0)),
            pl.BlockSpec((G, C, E), lambda e: (0, 0, 0)),
            pl.BlockSpec((1, 3, 3, C, CCAP), lambda e: (e, 0, 0, 0, 0)),
            pl.BlockSpec((1, 1, CCAP), lambda e: (e, 0, 0)),
            pl.BlockSpec((1, CCAP, C), lambda e: (e, 0, 0)),
            pl.BlockSpec((1, 1, C), lambda e: (e, 0, 0)),
        ],
        out_specs=[
            pl.BlockSpec((G, BHW, C), lambda e: (0, 0, 0)),
            pl.BlockSpec((1, 1), lambda e: (0, 0)),
        ],
        out_shape=[
            jax.ShapeDtypeStruct((G, BHW, C), jnp.float32),
            jax.ShapeDtypeStruct((1, 1), jnp.float32),
        ],
        scratch_shapes=[pltpu.VMEM((G, E, B, 1), jnp.float32)],
        compiler_params=pltpu.CompilerParams(
            dimension_semantics=("arbitrary",),
        ),
    )(x3, x3b, gates, Wc_r, bc_r, Wp_r, bp_r)

    ys4 = jnp.transpose(ys.reshape(G, B, H, W, C), (0, 1, 4, 2, 3))
    return ys4[0], ys4[1], ys4[2], ys4[3], loss[0, 0]


def kernel(x, Wc, bc, Wp, bp, gates):
    return _moe(x, Wc, bc, Wp, bp, gates)
